# baseline (speedup 1.0000x reference)
import dataclasses
import functools

import numpy as np
import jax
import jax.numpy as jnp
from jax import lax
from jax.experimental import pallas as pl
from jax.experimental.pallas import tpu as pltpu
from jax.experimental.pallas import tpu_sc as plsc

_D = 64
_SEQ = 16384
_NC = 2
_NS = 16
_L = 16
_NW = _NC * _NS
_BPW = _SEQ // _NW
_VP = 500000
_CHUNK = 128
_NCHUNK = _BPW // _CHUNK


def _pe_table() -> np.ndarray:
    i = np.arange(_SEQ, dtype=np.float32)[:, None]
    j = np.arange(_D, dtype=np.float32)[None, :]
    angle = i / np.power(np.float32(10000.0), j / np.float32(_D))
    even = (np.arange(_D)[None, :] % 2) == 0
    return np.where(even, np.sin(angle), np.cos(angle)).astype(np.float32)


_PE = _pe_table()


def _compiler_params():
    cp = pltpu.CompilerParams()
    if "needs_layout_passes" in pltpu.CompilerParams.__dataclass_fields__:
        cp = dataclasses.replace(cp, needs_layout_passes=False)
    return cp


def _sc_embed(table2, x, pe):
    mesh = plsc.VectorSubcoreMesh(core_axis_name="c", subcore_axis_name="s")

    @functools.partial(
        pl.kernel,
        out_type=jax.ShapeDtypeStruct((_SEQ, _D), jnp.float32),
        mesh=mesh,
        scratch_types=[
            pltpu.VMEM((_BPW,), jnp.int32),
            pltpu.VMEM((_BPW,), jnp.int32),
            pltpu.VMEM((_CHUNK, 2 * _D), jnp.float32),
            pltpu.VMEM((_CHUNK, 2 * _D), jnp.float32),
            pltpu.VMEM((_BPW, _D), jnp.float32),
            pltpu.SemaphoreType.DMA,
            pltpu.SemaphoreType.DMA,
            pltpu.SemaphoreType.DMA,
        ],
        compiler_params=_compiler_params(),
    )
    def k(tab_hbm, x_hbm, pe_hbm, out_hbm, idx_v, qidx_v, gat_a, gat_b, pe_v,
          gsem0, gsem1, psem):
        wid = lax.axis_index("s") * _NC + lax.axis_index("c")
        base = wid * _BPW
        pltpu.sync_copy(x_hbm.at[pl.ds(base, _BPW)], idx_v)
        pe_cp = pltpu.async_copy(pe_hbm.at[pl.ds(base, _BPW)], pe_v, psem)

        @pl.loop(0, _BPW, step=4 * _L)
        def _(j):
            for u in range(4):
                s = pl.ds(j + u * _L, _L)
                qidx_v.at[s][...] = lax.shift_right_logical(idx_v.at[s][...], 1)

        bufs = (gat_a, gat_b)
        sems = (gsem0, gsem1)

        def fire(b):
            s = pl.ds(b * _CHUNK, _CHUNK)
            return pltpu.async_copy(
                tab_hbm.at[qidx_v.at[s]], bufs[b % 2], sems[b % 2]
            )

        def add_batch(b):
            buf = bufs[b % 2]

            @pl.loop(0, _CHUNK, step=_L)
            def _(g):
                lanes = lax.iota(jnp.int32, _L)
                xv = idx_v.at[pl.ds(b * _CHUNK + g, _L)][...]
                offv = (xv & 1) * _D
                grow = lanes + g
                prow = lanes + b * _CHUNK + g
                for j in range(_D):
                    a = plsc.load_gather(buf, [grow, offv + j])
                    pv = plsc.load_gather(pe_v, [prow, lanes * 0 + j])
                    plsc.store_scatter(pe_v, [prow, lanes * 0 + j], a + pv)

        cps = [fire(0), fire(1)]
        pe_cp.wait()
        for b in range(_NCHUNK):
            cps[b].wait()
            add_batch(b)
            if b + 2 < _NCHUNK:
                cps.append(fire(b + 2))

        pltpu.sync_copy(pe_v, out_hbm.at[pl.ds(base, _BPW)])

    return k(table2, x, pe)


def kernel(x, table):
    return _sc_embed(jnp.reshape(table, (_VP, 2 * _D)), x, _PE)


# speedup vs baseline: 2.6963x; 2.6963x over previous
import dataclasses
import functools

import numpy as np
import jax
import jax.numpy as jnp
from jax import lax
from jax.experimental import pallas as pl
from jax.experimental.pallas import tpu as pltpu
from jax.experimental.pallas import tpu_sc as plsc

_D = 64
_SEQ = 16384
_NC = 2
_NS = 16
_L = 16
_NW = _NC * _NS
_BPW = _SEQ // _NW
_NGRP = _BPW // _L
_NBUF = 4


def _pe_table() -> np.ndarray:
    i = np.arange(_SEQ, dtype=np.float32)[:, None]
    j = np.arange(_D, dtype=np.float32)[None, :]
    angle = i / np.power(np.float32(10000.0), j / np.float32(_D))
    even = (np.arange(_D)[None, :] % 2) == 0
    return np.where(even, np.sin(angle), np.cos(angle)).astype(np.float32)


_PE = _pe_table()


def _compiler_params():
    cp = pltpu.CompilerParams()
    if "needs_layout_passes" in pltpu.CompilerParams.__dataclass_fields__:
        cp = dataclasses.replace(cp, needs_layout_passes=False)
    return cp


def _sc_embed(table_t, x, pe):
    mesh = plsc.VectorSubcoreMesh(core_axis_name="c", subcore_axis_name="s")

    @functools.partial(
        pl.kernel,
        out_type=jax.ShapeDtypeStruct((_SEQ, _D), jnp.float32),
        mesh=mesh,
        scratch_types=[
            pltpu.VMEM((_BPW,), jnp.int32),
            pltpu.VMEM((_NBUF, _D, 2 * _D), jnp.float32),
            pltpu.VMEM((_BPW, _D), jnp.float32),
        ]
        + [pltpu.SemaphoreType.DMA] * (_NBUF + 1),
        compiler_params=_compiler_params(),
    )
    def k(tab_hbm, x_hbm, pe_hbm, out_hbm, idx_v, ring_v, pe_v, *sems):
        bsems, psem = sems[:_NBUF], sems[_NBUF]
        wid = lax.axis_index("s") * _NC + lax.axis_index("c")
        base = wid * _BPW
        pltpu.sync_copy(x_hbm.at[pl.ds(base, _BPW)], idx_v)
        pe_cp = pltpu.async_copy(pe_hbm.at[pl.ds(base, _BPW)], pe_v, psem)

        def fire(slot, xs):
            q128 = pl.multiple_of(lax.shift_right_logical(xs, 7) * 128, 128)
            pltpu.async_copy(
                tab_hbm.at[:, pl.ds(q128, 128)], ring_v.at[slot], bsems[slot]
            )

        def wait(slot):
            pltpu.make_async_copy(
                tab_hbm.at[:, pl.ds(0, 128)], ring_v.at[slot], bsems[slot]
            ).wait()

        xv0 = idx_v.at[pl.ds(0, _L)][...]
        for u in range(_NBUF):
            fire(u, xv0[u])
        pe_cp.wait()

        @pl.loop(0, _NGRP)
        def _(g):
            i0 = g * _L
            xv = idx_v.at[pl.ds(i0, _L)][...]
            nxt = jnp.minimum((g + 1) * _L, _BPW - _L)
            xn = idx_v.at[pl.ds(nxt, _L)][...]
            lanes = lax.iota(jnp.int32, _L)
            for u in range(_L):
                slot = u % _NBUF
                wait(slot)
                o = xv[u] & 127
                row = i0 + u
                for c0 in range(0, _D, _L):
                    a = plsc.load_gather(
                        ring_v.at[slot], [lanes + c0, lanes * 0 + o]
                    )
                    s = (row, pl.ds(c0, _L))
                    pe_v.at[s][...] = pe_v.at[s][...] + a
                xnext = xv[u + _NBUF] if u + _NBUF < _L else xn[u + _NBUF - _L]
                fire(slot, xnext)

        for u in range(_NBUF):
            wait(u % _NBUF)

        pltpu.sync_copy(pe_v, out_hbm.at[pl.ds(base, _BPW)])

    return k(table_t, x, pe)


def kernel(x, table):
    return _sc_embed(table.T, x, _PE)
